# baseline (speedup 1.0000x reference)
import functools

import jax
import jax.numpy as jnp
from jax import lax
from jax.experimental import pallas as pl
from jax.experimental.pallas import tpu as pltpu
from jax.experimental.pallas import tpu_sc as plsc

B = 16384
DM = 32
NC = 2
NS = 16
NW = NC * NS
BPW = B // NW
CHUNK = 128
NCHUNK = BPW // CHUNK


def _pmf_body(ui_hbm, ii_hbm, u_hbm, v_hbm, c_hbm, d_hbm, out_hbm,
              uidx, iidx, ub, vb, cb, db, ob, sem):
    wid = lax.axis_index("s") * NC + lax.axis_index("c")
    base = wid * BPW

    pltpu.sync_copy(ui_hbm.at[pl.ds(base, BPW)], uidx)
    pltpu.sync_copy(ii_hbm.at[pl.ds(base, BPW)], iidx)

    def chunk_body(j, _):
        us = uidx.at[pl.ds(j * CHUNK, CHUNK)]
        it = iidx.at[pl.ds(j * CHUNK, CHUNK)]
        copies = []
        for k in range(DM):
            copies.append(pltpu.async_copy(u_hbm.at[k].at[us], ub.at[k], sem))
            copies.append(pltpu.async_copy(v_hbm.at[k].at[it], vb.at[k], sem))
            copies.append(pltpu.async_copy(c_hbm.at[k].at[us], cb.at[k], sem))
            copies.append(pltpu.async_copy(d_hbm.at[k].at[it], db.at[k], sem))
        for cp in copies:
            cp.wait()

        for c in range(CHUNK // 16):
            col = pl.ds(c * 16, 16)
            acc = ub[0, col] * vb[0, col] + cb[0, col] * db[0, col]
            for k in range(1, DM):
                acc = acc + ub[k, col] * vb[k, col] + cb[k, col] * db[k, col]
            ob[pl.ds(j * CHUNK + c * 16, 16)] = acc
        return 0

    lax.fori_loop(0, NCHUNK, chunk_body, 0)

    pltpu.sync_copy(ob, out_hbm.at[pl.ds(base, BPW)])


@jax.jit
def _pmf(users_index, items_index, ut, vt, ct, dt):
    mesh = plsc.VectorSubcoreMesh(core_axis_name="c", subcore_axis_name="s")
    f = functools.partial(
        pl.kernel,
        mesh=mesh,
        compiler_params=pltpu.CompilerParams(
            needs_layout_passes=False, use_tc_tiling_on_sc=False),
        out_type=jax.ShapeDtypeStruct((B,), jnp.float32),
        scratch_types=[
            pltpu.VMEM((BPW,), jnp.int32),
            pltpu.VMEM((BPW,), jnp.int32),
            pltpu.VMEM((DM, CHUNK), jnp.float32),
            pltpu.VMEM((DM, CHUNK), jnp.float32),
            pltpu.VMEM((DM, CHUNK), jnp.float32),
            pltpu.VMEM((DM, CHUNK), jnp.float32),
            pltpu.VMEM((BPW,), jnp.float32),
            pltpu.SemaphoreType.DMA,
        ],
    )(_pmf_body)
    return f(users_index, items_index, ut, vt, ct, dt)


def kernel(users_index, items_index, U, V, C, D):
    return _pmf(users_index.astype(jnp.int32), items_index,
                U.T, V.T, C.T, D.T)


# speedup vs baseline: 6.1617x; 6.1617x over previous
import functools

import jax
import jax.numpy as jnp
from jax import lax
from jax.experimental import pallas as pl
from jax.experimental.pallas import tpu as pltpu
from jax.experimental.pallas import tpu_sc as plsc

B = 16384
DM = 32
NC = 2
NS = 16
NW = NC * NS
BPW = B // NW
CHUNK = 128
NCHUNK = BPW // CHUNK


def _pmf_body(ui_hbm, ii_hbm, u_hbm, v_hbm, c_hbm, d_hbm, out_hbm,
              uidx, iidx, ub, vb, cb, db, ob, sem):
    wid = lax.axis_index("s") * NC + lax.axis_index("c")
    base = wid * BPW

    pltpu.sync_copy(ui_hbm.at[pl.ds(base, BPW)], uidx)
    pltpu.sync_copy(ii_hbm.at[pl.ds(base, BPW)], iidx)

    copies = []
    for j in range(NCHUNK):
        us = uidx.at[pl.ds(j * CHUNK, CHUNK)]
        it = iidx.at[pl.ds(j * CHUNK, CHUNK)]
        row = pl.ds(j * CHUNK, CHUNK)
        copies.append(pltpu.async_copy(u_hbm.at[us], ub.at[row], sem))
        copies.append(pltpu.async_copy(v_hbm.at[it], vb.at[row], sem))
        copies.append(pltpu.async_copy(c_hbm.at[us], cb.at[row], sem))
        copies.append(pltpu.async_copy(d_hbm.at[it], db.at[row], sem))
    for cp in copies:
        cp.wait()

    lane = lax.iota(jnp.int32, 16)

    def group(g, _):
        rows = g * 16 + lane
        acc = jnp.zeros((16,), jnp.float32)
        for k in range(DM):
            col = (lane + k) & (DM - 1)
            pu = plsc.load_gather(ub, [rows, col])
            pv = plsc.load_gather(vb, [rows, col])
            pc = plsc.load_gather(cb, [rows, col])
            pd = plsc.load_gather(db, [rows, col])
            acc = acc + pu * pv + pc * pd
        ob[pl.ds(g * 16, 16)] = acc
        return 0

    lax.fori_loop(0, BPW // 16, group, 0)

    pltpu.sync_copy(ob, out_hbm.at[pl.ds(base, BPW)])


@jax.jit
def _pmf(users_index, items_index, U, V, C, D):
    mesh = plsc.VectorSubcoreMesh(core_axis_name="c", subcore_axis_name="s")
    f = functools.partial(
        pl.kernel,
        mesh=mesh,
        compiler_params=pltpu.CompilerParams(
            needs_layout_passes=False, use_tc_tiling_on_sc=False),
        out_type=jax.ShapeDtypeStruct((B,), jnp.float32),
        scratch_types=[
            pltpu.VMEM((BPW,), jnp.int32),
            pltpu.VMEM((BPW,), jnp.int32),
            pltpu.VMEM((BPW, DM), jnp.float32),
            pltpu.VMEM((BPW, DM), jnp.float32),
            pltpu.VMEM((BPW, DM), jnp.float32),
            pltpu.VMEM((BPW, DM), jnp.float32),
            pltpu.VMEM((BPW,), jnp.float32),
            pltpu.SemaphoreType.DMA,
        ],
    )(_pmf_body)
    return f(users_index, items_index, U, V, C, D)


def kernel(users_index, items_index, U, V, C, D):
    return _pmf(users_index.astype(jnp.int32), items_index, U, V, C, D)


# speedup vs baseline: 19.7810x; 3.2103x over previous
import functools

import jax
import jax.numpy as jnp
from jax import lax
from jax.experimental import pallas as pl
from jax.experimental.pallas import tpu as pltpu
from jax.experimental.pallas import tpu_sc as plsc

B = 16384
DM = 32
NC = 2
NS = 16
NW = NC * NS
BPW = B // NW
CHUNK = 128
NCHUNK = BPW // CHUNK
NT = 7813
ROW = NT * 128
SW = 13
NSTRIPE = NT // SW
SWW = SW * 128


def _detile_body(u_hbm, v_hbm, c_hbm, d_hbm, uo, vo, co, do_, buf, sem, semw):
    wid = lax.axis_index("s") * NC + lax.axis_index("c")

    def stripe(n, _):
        s = wid + NW * n

        @pl.when(s < NSTRIPE)
        def _():
            col = pl.ds(s * SWW, SWW)
            for t_hbm, t_out in ((u_hbm, uo), (v_hbm, vo),
                                 (c_hbm, co), (d_hbm, do_)):
                pltpu.async_copy(t_hbm.at[:, col], buf, sem).wait()
                wcopies = []
                for k in range(DM):
                    dst = t_out.at[pl.ds(k * ROW + s * SWW, SWW)]
                    wcopies.append(pltpu.async_copy(buf.at[k], dst, semw))
                for cp in wcopies:
                    cp.wait()
        return 0

    lax.fori_loop(0, (NSTRIPE + NW - 1) // NW, stripe, 0)


def _gather_body(ui_hbm, ii_hbm, uf, vf, cf, df, out_hbm,
                 uidx, iidx, ub, vb, cb, db, ob, sem):
    wid = lax.axis_index("s") * NC + lax.axis_index("c")
    base = wid * BPW

    pltpu.sync_copy(ui_hbm.at[pl.ds(base, BPW)], uidx)
    pltpu.sync_copy(ii_hbm.at[pl.ds(base, BPW)], iidx)

    def chunk_body(j, _):
        us = uidx.at[pl.ds(j * CHUNK, CHUNK)]
        it = iidx.at[pl.ds(j * CHUNK, CHUNK)]
        copies = []
        for k in range(DM):
            row = pl.ds(k * ROW, ROW)
            copies.append(pltpu.async_copy(uf.at[row].at[us], ub.at[k], sem))
            copies.append(pltpu.async_copy(vf.at[row].at[it], vb.at[k], sem))
            copies.append(pltpu.async_copy(cf.at[row].at[us], cb.at[k], sem))
            copies.append(pltpu.async_copy(df.at[row].at[it], db.at[k], sem))
        for cp in copies:
            cp.wait()

        for c in range(CHUNK // 16):
            col = pl.ds(c * 16, 16)
            acc = ub[0, col] * vb[0, col] + cb[0, col] * db[0, col]
            for k in range(1, DM):
                acc = acc + ub[k, col] * vb[k, col] + cb[k, col] * db[k, col]
            ob[pl.ds(j * CHUNK + c * 16, 16)] = acc
        return 0

    lax.fori_loop(0, NCHUNK, chunk_body, 0)

    pltpu.sync_copy(ob, out_hbm.at[pl.ds(base, BPW)])


@jax.jit
def _pmf(users_index, items_index, ut, vt, ct, dt):
    mesh = plsc.VectorSubcoreMesh(core_axis_name="c", subcore_axis_name="s")
    flat = jax.ShapeDtypeStruct((DM * ROW,), jnp.float32)

    detile = functools.partial(
        pl.kernel,
        mesh=mesh,
        compiler_params=pltpu.CompilerParams(
            needs_layout_passes=False, use_tc_tiling_on_sc=True),
        out_type=(flat, flat, flat, flat),
        scratch_types=[
            pltpu.VMEM((DM, SWW), jnp.float32),
            pltpu.SemaphoreType.DMA,
            pltpu.SemaphoreType.DMA,
        ],
    )(_detile_body)
    uf, vf, cf, df = detile(ut, vt, ct, dt)

    gather = functools.partial(
        pl.kernel,
        mesh=mesh,
        compiler_params=pltpu.CompilerParams(
            needs_layout_passes=False, use_tc_tiling_on_sc=False),
        out_type=jax.ShapeDtypeStruct((B,), jnp.float32),
        scratch_types=[
            pltpu.VMEM((BPW,), jnp.int32),
            pltpu.VMEM((BPW,), jnp.int32),
            pltpu.VMEM((DM, CHUNK), jnp.float32),
            pltpu.VMEM((DM, CHUNK), jnp.float32),
            pltpu.VMEM((DM, CHUNK), jnp.float32),
            pltpu.VMEM((DM, CHUNK), jnp.float32),
            pltpu.VMEM((BPW,), jnp.float32),
            pltpu.SemaphoreType.DMA,
        ],
    )(_gather_body)
    return gather(users_index, items_index, uf, vf, cf, df)


def kernel(users_index, items_index, U, V, C, D):
    return _pmf(users_index.astype(jnp.int32), items_index,
                U.T, V.T, C.T, D.T)


# speedup vs baseline: 21.6077x; 1.0923x over previous
import functools

import jax
import jax.numpy as jnp
from jax import lax
from jax.experimental import pallas as pl
from jax.experimental.pallas import tpu as pltpu
from jax.experimental.pallas import tpu_sc as plsc

B = 16384
DM = 32
NC = 2
NS = 16
NW = NC * NS
BPW = B // NW
CHUNK = 128
NCHUNK = BPW // CHUNK
NT = 7813
ROW = NT * 128
SW = 13
NSTRIPE = NT // SW
SWW = SW * 128


def _detile_body(u_hbm, v_hbm, c_hbm, d_hbm, uo, vo, co, do_,
                 buf0, buf1, semr, semw):
    wid = lax.axis_index("s") * NC + lax.axis_index("c")
    bufs = (buf0, buf1)
    tables = ((u_hbm, uo), (v_hbm, vo), (c_hbm, co), (d_hbm, do_))

    def stripe(n, _):
        s = wid + NW * n

        @pl.when(s < NSTRIPE)
        def _():
            col = pl.ds(s * SWW, SWW)
            reads = [pltpu.async_copy(tables[0][0].at[:, col], bufs[0], semr)]
            writes = []
            for t in range(4):
                reads[t].wait()
                if t >= 1:
                    for cp in writes[t - 1]:
                        cp.wait()
                if t < 3:
                    reads.append(pltpu.async_copy(
                        tables[t + 1][0].at[:, col], bufs[(t + 1) % 2], semr))
                t_out = tables[t][1]
                wcopies = []
                for k in range(DM):
                    dst = t_out.at[pl.ds(k * ROW + s * SWW, SWW)]
                    wcopies.append(pltpu.async_copy(bufs[t % 2].at[k], dst,
                                                    semw))
                writes.append(wcopies)
            for cp in writes[3]:
                cp.wait()
        return 0

    lax.fori_loop(0, (NSTRIPE + NW - 1) // NW, stripe, 0)


def _gather_body(ui_hbm, ii_hbm, uf, vf, cf, df, out_hbm,
                 uidx, iidx, ub, vb, cb, db, ob, sem):
    wid = lax.axis_index("s") * NC + lax.axis_index("c")
    base = wid * BPW

    pltpu.sync_copy(ui_hbm.at[pl.ds(base, BPW)], uidx)
    pltpu.sync_copy(ii_hbm.at[pl.ds(base, BPW)], iidx)

    def chunk_body(j, _):
        us = uidx.at[pl.ds(j * CHUNK, CHUNK)]
        it = iidx.at[pl.ds(j * CHUNK, CHUNK)]
        copies = []
        for k in range(DM):
            row = pl.ds(k * ROW, ROW)
            copies.append(pltpu.async_copy(uf.at[row].at[us], ub.at[k], sem))
            copies.append(pltpu.async_copy(vf.at[row].at[it], vb.at[k], sem))
            copies.append(pltpu.async_copy(cf.at[row].at[us], cb.at[k], sem))
            copies.append(pltpu.async_copy(df.at[row].at[it], db.at[k], sem))
        for cp in copies:
            cp.wait()

        for c in range(CHUNK // 16):
            col = pl.ds(c * 16, 16)
            acc = ub[0, col] * vb[0, col] + cb[0, col] * db[0, col]
            for k in range(1, DM):
                acc = acc + ub[k, col] * vb[k, col] + cb[k, col] * db[k, col]
            ob[pl.ds(j * CHUNK + c * 16, 16)] = acc
        return 0

    lax.fori_loop(0, NCHUNK, chunk_body, 0)

    pltpu.sync_copy(ob, out_hbm.at[pl.ds(base, BPW)])


@jax.jit
def _pmf(users_index, items_index, ut, vt, ct, dt):
    mesh = plsc.VectorSubcoreMesh(core_axis_name="c", subcore_axis_name="s")
    flat = jax.ShapeDtypeStruct((DM * ROW,), jnp.float32)

    detile = functools.partial(
        pl.kernel,
        mesh=mesh,
        compiler_params=pltpu.CompilerParams(
            needs_layout_passes=False, use_tc_tiling_on_sc=True),
        out_type=(flat, flat, flat, flat),
        scratch_types=[
            pltpu.VMEM((DM, SWW), jnp.float32),
            pltpu.VMEM((DM, SWW), jnp.float32),
            pltpu.SemaphoreType.DMA,
            pltpu.SemaphoreType.DMA,
        ],
    )(_detile_body)
    uf, vf, cf, df = detile(ut, vt, ct, dt)

    gather = functools.partial(
        pl.kernel,
        mesh=mesh,
        compiler_params=pltpu.CompilerParams(
            needs_layout_passes=False, use_tc_tiling_on_sc=False),
        out_type=jax.ShapeDtypeStruct((B,), jnp.float32),
        scratch_types=[
            pltpu.VMEM((BPW,), jnp.int32),
            pltpu.VMEM((BPW,), jnp.int32),
            pltpu.VMEM((DM, CHUNK), jnp.float32),
            pltpu.VMEM((DM, CHUNK), jnp.float32),
            pltpu.VMEM((DM, CHUNK), jnp.float32),
            pltpu.VMEM((DM, CHUNK), jnp.float32),
            pltpu.VMEM((BPW,), jnp.float32),
            pltpu.SemaphoreType.DMA,
        ],
    )(_gather_body)
    return gather(users_index, items_index, uf, vf, cf, df)


def kernel(users_index, items_index, U, V, C, D):
    return _pmf(users_index.astype(jnp.int32), items_index,
                U.T, V.T, C.T, D.T)
